# baseline (speedup 1.0000x reference)
import functools

import jax
import jax.numpy as jnp
import numpy as np
from jax import lax
from jax.experimental import pallas as pl
from jax.experimental.pallas import tpu as pltpu
from jax.experimental.pallas import tpu_sc as plsc

_NUM_BINS = 16
_GAMMA = 0.01
_HIDDEN = 32
_K = 9
_B, _C, _H, _W = 4, 3, 512, 512
_NIMG = _B * _C
_PIX_PER_IMG = _H * _W
_ROWS = _NIMG * _H
_NWORKERS = 32
_TROWS = 64
_TILE = _TROWS * _W
_NSUB = _ROWS // (_NWORKERS * _TROWS)
_LANES = 16
_TAB = 2 * _NIMG * (_K - 1)

_INV_DT = np.float32(1.0 / (0.125 + 1e-8))


def _hist_mlp_body(x_ref, w1_ref, b1_ref, w2_ref, b2_ref, tab_ref, hist_ref,
                   acc_ref):
    b = pl.program_id(0)
    xb = x_ref[0]
    inv_g = 1.0 / _GAMMA
    zc = 2.0 * inv_g / (_NUM_BINS - 1)
    half = _NUM_BINS // 2
    cr = 64
    ones_row = jnp.ones((1, cr), jnp.bfloat16)
    for ch in range(_H // cr):
        xs32 = xb[ch * cr:(ch + 1) * cr, :]
        for lo in (True, False):
            xsv = xs32 if lo else 1.0 - xs32
            p = jnp.exp(xsv * xsv * (-inv_g))
            z = jnp.exp(xsv * zc)
            zb = z.astype(jnp.bfloat16)
            w = p.astype(jnp.bfloat16)
            for k in range(half):
                if k:
                    w = w * zb
                row = lax.dot_general(ones_row, w, (((1,), (0,)), ((), ())),
                                      preferred_element_type=jnp.float32)
                tgt = k if lo else _NUM_BINS - 1 - k
                if ch == 0:
                    acc_ref[pl.ds(tgt, 1), :] = row
                else:
                    acc_ref[pl.ds(tgt, 1), :] = acc_ref[pl.ds(tgt, 1), :] + row
    acc = acc_ref[...]
    cc = lax.broadcasted_iota(jnp.int32, (_NUM_BINS, 1), 0).astype(
        jnp.float32) * (1.0 / (_NUM_BINS - 1))
    cc = jnp.minimum(cc, 1.0 - cc)
    amp = jnp.exp(cc * cc * (-inv_g))
    hrow = (jnp.sum(acc * amp, axis=1) * (1.0 / _PIX_PER_IMG)).reshape(
        1, _NUM_BINS)
    hist_ref[pl.ds(b, 1), :] = hrow

    @pl.when(b == _NIMG - 1)
    def _():
        h = hist_ref[...]
        hid = lax.dot_general(h, w1_ref[...], (((1,), (1,)), ((), ())),
                              preferred_element_type=jnp.float32)
        hid = jnp.maximum(hid + b1_ref[...], 0.0)
        raw = lax.dot_general(hid, w2_ref[...], (((1,), (1,)), ((), ())),
                              preferred_element_type=jnp.float32)
        raw = raw + b2_ref[...]
        sp = jnp.maximum(raw, 0.0) + jnp.log1p(jnp.exp(-jnp.abs(raw)))
        inc = sp + 0.001
        inc = inc / jnp.sum(inc, axis=1, keepdims=True)
        ii = lax.broadcasted_iota(jnp.int32, (_K - 1, _K - 1), 0)
        jj = lax.broadcasted_iota(jnp.int32, (_K - 1, _K - 1), 1)
        excl = (jj > ii).astype(jnp.float32)
        klo = lax.dot_general(inc, excl, (((1,), (0,)), ((), ())),
                              preferred_element_type=jnp.float32)
        t_lo = lax.broadcasted_iota(jnp.int32, (1, _K - 1), 1).astype(
            jnp.float32) * 0.125
        slope = inc * _INV_DT
        icept = klo - t_lo * slope
        both = jnp.concatenate([slope, icept], axis=0)
        for rr in range(2 * _NIMG):
            tab_ref[pl.ds(rr * (_K - 1), _K - 1)] = both[rr]


def _hist_mlp_tc(x12, W1, b1r, W2, b2r):
    return pl.pallas_call(
        _hist_mlp_body,
        grid=(_NIMG,),
        in_specs=[
            pl.BlockSpec((1, _H, _W), lambda b: (b, 0, 0)),
            pl.BlockSpec((_HIDDEN, _NUM_BINS), lambda b: (0, 0)),
            pl.BlockSpec((1, _HIDDEN), lambda b: (0, 0)),
            pl.BlockSpec((_K - 1, _HIDDEN), lambda b: (0, 0)),
            pl.BlockSpec((1, _K - 1), lambda b: (0, 0)),
        ],
        out_specs=pl.BlockSpec((_TAB,), lambda b: (0,)),
        out_shape=jax.ShapeDtypeStruct((_TAB,), jnp.float32),
        scratch_shapes=[pltpu.VMEM((_NIMG, _NUM_BINS), jnp.float32),
                        pltpu.VMEM((_NUM_BINS, _W), jnp.float32)],
    )(x12, W1, b1r, W2, b2r)


@functools.cache
def _make_lut_sc():
    return functools.partial(
        pl.kernel,
        out_type=jax.ShapeDtypeStruct((_ROWS, _W), jnp.float32),
        mesh=plsc.VectorSubcoreMesh(core_axis_name="c", subcore_axis_name="s"),
        scratch_types=[
            pltpu.VMEM((_TROWS, _W), jnp.float32),
            pltpu.VMEM((_TROWS, _W), jnp.float32),
            pltpu.VMEM((_TROWS, _W), jnp.float32),
            pltpu.VMEM((_TAB,), jnp.float32),
            pltpu.SemaphoreType.DMA,
            pltpu.SemaphoreType.DMA,
            pltpu.SemaphoreType.DMA,
            pltpu.SemaphoreType.DMA,
            pltpu.SemaphoreType.DMA,
            pltpu.SemaphoreType.DMA,
        ],
        compiler_params=pltpu.CompilerParams(needs_layout_passes=False),
    )(_lut_sc_body)


def _lut_sc_body(x_hbm, tab_hbm, out_hbm, xb0, xb1, xb2, tabv,
                 si0, si1, si2, so0, so1, so2):
    wid = lax.axis_index("s") * 2 + lax.axis_index("c")
    pltpu.sync_copy(tab_hbm, tabv)
    xbs, sis, sos = (xb0, xb1, xb2), (si0, si1, si2), (so0, so1, so2)

    def row0(t):
        return (wid * _NSUB + t) * _TROWS

    in_cp = [pltpu.async_copy(x_hbm.at[pl.ds(row0(t), _TROWS), :],
                              xbs[t], sis[t]) for t in range(_NSUB)]
    out_cp = [None] * _NSUB
    for t in range(_NSUB):
        in_cp[t].wait()
        xb = xbs[t]
        img8 = lax.shift_right_logical(wid * _NSUB + t, 3) * (_K - 1)
        base = jnp.zeros((_LANES,), jnp.int32) + img8

        @plsc.parallel_loop(0, _TILE // _LANES, unroll=8)
        def body(i):
            r = lax.shift_right_logical(i, 5)
            c = (i & 31) * _LANES
            xv = xb[r, pl.ds(c, _LANES)]
            y = xv * 8.0
            idx = jnp.clip(y.astype(jnp.int32), 0, _K - 2)
            flat = base + idx
            s = plsc.load_gather(tabv, [flat])
            ic = plsc.load_gather(tabv, [flat + _NIMG * (_K - 1)])
            xb[r, pl.ds(c, _LANES)] = ic + s * xv
        out_cp[t] = pltpu.async_copy(
            xb, out_hbm.at[pl.ds(row0(t), _TROWS), :], sos[t])
    for t in range(_NSUB):
        out_cp[t].wait()


def kernel(x, W1, b1, W2, b2):
    x12 = x.reshape(_NIMG, _H, _W)
    tab = _hist_mlp_tc(x12, W1, b1.reshape(1, _HIDDEN), W2,
                       b2.reshape(1, _K - 1))
    y = _make_lut_sc()(x.reshape(_ROWS, _W), tab)
    return (y.reshape(_B, _C, _H, _W), jnp.zeros(()))


# speedup vs baseline: 1.0564x; 1.0564x over previous
import functools

import jax
import jax.numpy as jnp
import numpy as np
from jax import lax
from jax.experimental import pallas as pl
from jax.experimental.pallas import tpu as pltpu
from jax.experimental.pallas import tpu_sc as plsc

_NUM_BINS = 16
_GAMMA = 0.01
_HIDDEN = 32
_K = 9
_B, _C, _H, _W = 4, 3, 512, 512
_NIMG = _B * _C
_PIX_PER_IMG = _H * _W
_ROWS = _NIMG * _H
_NWORKERS = 32
_TROWS = 32
_TILE = _TROWS * _W
_NSUB = _ROWS // (_NWORKERS * _TROWS)
_LANES = 16
_TAB = 2 * _NIMG * (_K - 1)

_INV_DT = np.float32(1.0 / (0.125 + 1e-8))


def _hist_mlp_body(x_ref, w1_ref, b1_ref, w2_ref, b2_ref, tab_ref, hist_ref,
                   acc_ref):
    b = pl.program_id(0)
    xb = x_ref[0]
    inv_g = 1.0 / _GAMMA
    zc = 2.0 * inv_g / (_NUM_BINS - 1)
    half = _NUM_BINS // 2
    cr = 64
    ones_row = jnp.ones((1, cr), jnp.bfloat16)
    for ch in range(_H // cr):
        xs32 = xb[ch * cr:(ch + 1) * cr, :]
        for lo in (True, False):
            xsv = xs32 if lo else 1.0 - xs32
            p = jnp.exp(xsv * xsv * (-inv_g))
            z = jnp.exp(xsv * zc)
            zb = z.astype(jnp.bfloat16)
            w = p.astype(jnp.bfloat16)
            for k in range(half):
                if k:
                    w = w * zb
                row = lax.dot_general(ones_row, w, (((1,), (0,)), ((), ())),
                                      preferred_element_type=jnp.float32)
                tgt = k if lo else _NUM_BINS - 1 - k
                if ch == 0:
                    acc_ref[pl.ds(tgt, 1), :] = row
                else:
                    acc_ref[pl.ds(tgt, 1), :] = acc_ref[pl.ds(tgt, 1), :] + row
    acc = acc_ref[...]
    cc = lax.broadcasted_iota(jnp.int32, (_NUM_BINS, 1), 0).astype(
        jnp.float32) * (1.0 / (_NUM_BINS - 1))
    cc = jnp.minimum(cc, 1.0 - cc)
    amp = jnp.exp(cc * cc * (-inv_g))
    hrow = (jnp.sum(acc * amp, axis=1) * (1.0 / _PIX_PER_IMG)).reshape(
        1, _NUM_BINS)
    hist_ref[pl.ds(b, 1), :] = hrow

    @pl.when(b == _NIMG - 1)
    def _():
        h = hist_ref[...]
        hid = lax.dot_general(h, w1_ref[...], (((1,), (1,)), ((), ())),
                              preferred_element_type=jnp.float32)
        hid = jnp.maximum(hid + b1_ref[...], 0.0)
        raw = lax.dot_general(hid, w2_ref[...], (((1,), (1,)), ((), ())),
                              preferred_element_type=jnp.float32)
        raw = raw + b2_ref[...]
        sp = jnp.maximum(raw, 0.0) + jnp.log1p(jnp.exp(-jnp.abs(raw)))
        inc = sp + 0.001
        inc = inc / jnp.sum(inc, axis=1, keepdims=True)
        ii = lax.broadcasted_iota(jnp.int32, (_K - 1, _K - 1), 0)
        jj = lax.broadcasted_iota(jnp.int32, (_K - 1, _K - 1), 1)
        excl = (jj > ii).astype(jnp.float32)
        klo = lax.dot_general(inc, excl, (((1,), (0,)), ((), ())),
                              preferred_element_type=jnp.float32)
        t_lo = lax.broadcasted_iota(jnp.int32, (1, _K - 1), 1).astype(
            jnp.float32) * 0.125
        slope = inc * _INV_DT
        icept = klo - t_lo * slope
        both = jnp.concatenate([slope, icept], axis=0)
        for rr in range(2 * _NIMG):
            tab_ref[pl.ds(rr * (_K - 1), _K - 1)] = both[rr]


def _hist_mlp_tc(x12, W1, b1r, W2, b2r):
    return pl.pallas_call(
        _hist_mlp_body,
        grid=(_NIMG,),
        in_specs=[
            pl.BlockSpec((1, _H, _W), lambda b: (b, 0, 0)),
            pl.BlockSpec((_HIDDEN, _NUM_BINS), lambda b: (0, 0)),
            pl.BlockSpec((1, _HIDDEN), lambda b: (0, 0)),
            pl.BlockSpec((_K - 1, _HIDDEN), lambda b: (0, 0)),
            pl.BlockSpec((1, _K - 1), lambda b: (0, 0)),
        ],
        out_specs=pl.BlockSpec((_TAB,), lambda b: (0,)),
        out_shape=jax.ShapeDtypeStruct((_TAB,), jnp.float32),
        scratch_shapes=[pltpu.VMEM((_NIMG, _NUM_BINS), jnp.float32),
                        pltpu.VMEM((_NUM_BINS, _W), jnp.float32)],
    )(x12, W1, b1r, W2, b2r)


@functools.cache
def _make_lut_sc():
    return functools.partial(
        pl.kernel,
        out_type=jax.ShapeDtypeStruct((_ROWS, _W), jnp.float32),
        mesh=plsc.VectorSubcoreMesh(core_axis_name="c", subcore_axis_name="s"),
        scratch_types=[
            pltpu.VMEM((_TROWS, _W), jnp.float32),
            pltpu.VMEM((_TROWS, _W), jnp.float32),
            pltpu.VMEM((_TROWS, _W), jnp.float32),
            pltpu.VMEM((_TROWS, _W), jnp.float32),
            pltpu.VMEM((_TAB,), jnp.float32),
            pltpu.SemaphoreType.DMA,
            pltpu.SemaphoreType.DMA,
            pltpu.SemaphoreType.DMA,
            pltpu.SemaphoreType.DMA,
        ],
        compiler_params=pltpu.CompilerParams(needs_layout_passes=False),
    )(_lut_sc_body)


def _lut_sc_body(x_hbm, tab_hbm, out_hbm, xb0, xb1, ob0, ob1, tabv,
                 si0, si1, so0, so1):
    wid = lax.axis_index("s") * 2 + lax.axis_index("c")
    pltpu.sync_copy(tab_hbm, tabv)
    xbs, obs, sis, sos = (xb0, xb1), (ob0, ob1), (si0, si1), (so0, so1)

    def row0(t):
        return (wid * _NSUB + t) * _TROWS

    in_cp = [None] * _NSUB
    out_cp = [None] * _NSUB
    in_cp[0] = pltpu.async_copy(
        x_hbm.at[pl.ds(row0(0), _TROWS), :], xbs[0], sis[0])
    for t in range(_NSUB):
        pb = t % 2
        if t + 1 < _NSUB:
            in_cp[t + 1] = pltpu.async_copy(
                x_hbm.at[pl.ds(row0(t + 1), _TROWS), :], xbs[1 - pb],
                sis[1 - pb])
        in_cp[t].wait()
        if t >= 2:
            out_cp[t - 2].wait()
        xb, ob = xbs[pb], obs[pb]
        img8 = lax.shift_right_logical(wid * _NSUB + t, 4) * (_K - 1)
        base = jnp.zeros((_LANES,), jnp.int32) + img8

        @plsc.parallel_loop(0, _TILE // _LANES, unroll=8)
        def body(i):
            r = lax.shift_right_logical(i, 5)
            c = (i & 31) * _LANES
            xv = xb[r, pl.ds(c, _LANES)]
            y = xv * 8.0
            idx = jnp.clip(y.astype(jnp.int32), 0, _K - 2)
            flat = base + idx
            s = plsc.load_gather(tabv, [flat])
            ic = plsc.load_gather(tabv, [flat + _NIMG * (_K - 1)])
            ob[r, pl.ds(c, _LANES)] = ic + s * xv
        out_cp[t] = pltpu.async_copy(
            ob, out_hbm.at[pl.ds(row0(t), _TROWS), :], sos[pb])
    out_cp[_NSUB - 2].wait()
    out_cp[_NSUB - 1].wait()


def kernel(x, W1, b1, W2, b2):
    x12 = x.reshape(_NIMG, _H, _W)
    tab = _hist_mlp_tc(x12, W1, b1.reshape(1, _HIDDEN), W2,
                       b2.reshape(1, _K - 1))
    y = _make_lut_sc()(x.reshape(_ROWS, _W), tab)
    return (y.reshape(_B, _C, _H, _W), jnp.zeros(()))


# speedup vs baseline: 1.0786x; 1.0210x over previous
import functools

import jax
import jax.numpy as jnp
import numpy as np
from jax import lax
from jax.experimental import pallas as pl
from jax.experimental.pallas import tpu as pltpu
from jax.experimental.pallas import tpu_sc as plsc

_NUM_BINS = 16
_GAMMA = 0.01
_HIDDEN = 32
_K = 9
_B, _C, _H, _W = 4, 3, 512, 512
_NIMG = _B * _C
_PIX_PER_IMG = _H * _W
_ROWS = _NIMG * _H
_NWORKERS = 32
_TROWS = 32
_TILE = _TROWS * _W
_NSUB = _ROWS // (_NWORKERS * _TROWS)
_LANES = 16
_TAB = 2 * _NIMG * (_K - 1)

_INV_DT = np.float32(1.0 / (0.125 + 1e-8))


def _hist_mlp_body(x_ref, w1_ref, b1_ref, w2_ref, b2_ref, tab_ref, hist_ref,
                   acc_ref):
    b = pl.program_id(0)
    inv_g = 1.0 / _GAMMA
    zc = 2.0 * inv_g / (_NUM_BINS - 1)
    half = _NUM_BINS // 2
    cr = 64
    ones_row = jnp.ones((1, cr), jnp.bfloat16)
    for sub in range(2):
        xb = x_ref[0, sub * _H:(sub + 1) * _H, :]
        for ch in range(_H // cr):
            xs32 = xb[ch * cr:(ch + 1) * cr, :]
            for lo in (True, False):
                xsv = xs32 if lo else 1.0 - xs32
                p = jnp.exp(xsv * xsv * (-inv_g))
                z = jnp.exp(xsv * zc)
                zb = z.astype(jnp.bfloat16)
                w = p.astype(jnp.bfloat16)
                for k in range(half):
                    if k:
                        w = w * zb
                    row = lax.dot_general(ones_row, w,
                                          (((1,), (0,)), ((), ())),
                                          preferred_element_type=jnp.float32)
                    tgt = k if lo else _NUM_BINS - 1 - k
                    if ch == 0:
                        acc_ref[pl.ds(tgt, 1), :] = row
                    else:
                        acc_ref[pl.ds(tgt, 1), :] = (
                            acc_ref[pl.ds(tgt, 1), :] + row)
        acc = acc_ref[...]
        cc = lax.broadcasted_iota(jnp.int32, (_NUM_BINS, 1), 0).astype(
            jnp.float32) * (1.0 / (_NUM_BINS - 1))
        cc = jnp.minimum(cc, 1.0 - cc)
        amp = jnp.exp(cc * cc * (-inv_g))
        hrow = (jnp.sum(acc * amp, axis=1) * (1.0 / _PIX_PER_IMG)).reshape(
            1, _NUM_BINS)
        hist_ref[pl.ds(2 * b + sub, 1), :] = hrow

    @pl.when(b == _NIMG // 2 - 1)
    def _():
        h = hist_ref[...]
        hid = lax.dot_general(h, w1_ref[...], (((1,), (1,)), ((), ())),
                              preferred_element_type=jnp.float32)
        hid = jnp.maximum(hid + b1_ref[...], 0.0)
        raw = lax.dot_general(hid, w2_ref[...], (((1,), (1,)), ((), ())),
                              preferred_element_type=jnp.float32)
        raw = raw + b2_ref[...]
        sp = jnp.maximum(raw, 0.0) + jnp.log1p(jnp.exp(-jnp.abs(raw)))
        inc = sp + 0.001
        inc = inc / jnp.sum(inc, axis=1, keepdims=True)
        ii = lax.broadcasted_iota(jnp.int32, (_K - 1, _K - 1), 0)
        jj = lax.broadcasted_iota(jnp.int32, (_K - 1, _K - 1), 1)
        excl = (jj > ii).astype(jnp.float32)
        klo = lax.dot_general(inc, excl, (((1,), (0,)), ((), ())),
                              preferred_element_type=jnp.float32)
        t_lo = lax.broadcasted_iota(jnp.int32, (1, _K - 1), 1).astype(
            jnp.float32) * 0.125
        slope = inc * _INV_DT
        icept = klo - t_lo * slope
        both = jnp.concatenate([slope, icept], axis=0)
        for rr in range(2 * _NIMG):
            tab_ref[pl.ds(rr * (_K - 1), _K - 1)] = both[rr]


def _hist_mlp_tc(x12, W1, b1r, W2, b2r):
    return pl.pallas_call(
        _hist_mlp_body,
        grid=(_NIMG // 2,),
        in_specs=[
            pl.BlockSpec((1, 2 * _H, _W), lambda b: (b, 0, 0)),
            pl.BlockSpec((_HIDDEN, _NUM_BINS), lambda b: (0, 0)),
            pl.BlockSpec((1, _HIDDEN), lambda b: (0, 0)),
            pl.BlockSpec((_K - 1, _HIDDEN), lambda b: (0, 0)),
            pl.BlockSpec((1, _K - 1), lambda b: (0, 0)),
        ],
        out_specs=pl.BlockSpec((_TAB,), lambda b: (0,)),
        out_shape=jax.ShapeDtypeStruct((_TAB,), jnp.float32),
        scratch_shapes=[pltpu.VMEM((_NIMG, _NUM_BINS), jnp.float32),
                        pltpu.VMEM((_NUM_BINS, _W), jnp.float32)],
    )(x12, W1, b1r, W2, b2r)


@functools.cache
def _make_lut_sc():
    return functools.partial(
        pl.kernel,
        out_type=jax.ShapeDtypeStruct((_ROWS, _W), jnp.float32),
        mesh=plsc.VectorSubcoreMesh(core_axis_name="c", subcore_axis_name="s"),
        scratch_types=[
            pltpu.VMEM((_TROWS, _W), jnp.float32),
            pltpu.VMEM((_TROWS, _W), jnp.float32),
            pltpu.VMEM((_TROWS, _W), jnp.float32),
            pltpu.VMEM((_TROWS, _W), jnp.float32),
            pltpu.VMEM((_TAB,), jnp.float32),
            pltpu.SemaphoreType.DMA,
            pltpu.SemaphoreType.DMA,
            pltpu.SemaphoreType.DMA,
            pltpu.SemaphoreType.DMA,
        ],
        compiler_params=pltpu.CompilerParams(needs_layout_passes=False),
    )(_lut_sc_body)


def _lut_sc_body(x_hbm, tab_hbm, out_hbm, xb0, xb1, ob0, ob1, tabv,
                 si0, si1, so0, so1):
    wid = lax.axis_index("s") * 2 + lax.axis_index("c")
    pltpu.sync_copy(tab_hbm, tabv)
    xbs, obs, sis, sos = (xb0, xb1), (ob0, ob1), (si0, si1), (so0, so1)

    def row0(t):
        return (wid * _NSUB + t) * _TROWS

    in_cp = [None] * _NSUB
    out_cp = [None] * _NSUB
    in_cp[0] = pltpu.async_copy(
        x_hbm.at[pl.ds(row0(0), _TROWS), :], xbs[0], sis[0])
    for t in range(_NSUB):
        pb = t % 2
        if t + 1 < _NSUB:
            in_cp[t + 1] = pltpu.async_copy(
                x_hbm.at[pl.ds(row0(t + 1), _TROWS), :], xbs[1 - pb],
                sis[1 - pb])
        in_cp[t].wait()
        if t >= 2:
            out_cp[t - 2].wait()
        xb, ob = xbs[pb], obs[pb]
        img8 = lax.shift_right_logical(wid * _NSUB + t, 4) * (_K - 1)
        base = jnp.zeros((_LANES,), jnp.int32) + img8

        @plsc.parallel_loop(0, _TILE // _LANES, unroll=8)
        def body(i):
            r = lax.shift_right_logical(i, 5)
            c = (i & 31) * _LANES
            xv = xb[r, pl.ds(c, _LANES)]
            y = xv * 8.0
            idx = jnp.clip(y.astype(jnp.int32), 0, _K - 2)
            flat = base + idx
            s = plsc.load_gather(tabv, [flat])
            ic = plsc.load_gather(tabv, [flat + _NIMG * (_K - 1)])
            ob[r, pl.ds(c, _LANES)] = ic + s * xv
        out_cp[t] = pltpu.async_copy(
            ob, out_hbm.at[pl.ds(row0(t), _TROWS), :], sos[pb])
    out_cp[_NSUB - 2].wait()
    out_cp[_NSUB - 1].wait()


def kernel(x, W1, b1, W2, b2):
    x12 = x.reshape(_NIMG // 2, 2 * _H, _W)
    tab = _hist_mlp_tc(x12, W1, b1.reshape(1, _HIDDEN), W2,
                       b2.reshape(1, _K - 1))
    y = _make_lut_sc()(x.reshape(_ROWS, _W), tab)
    return (y.reshape(_B, _C, _H, _W), jnp.zeros(()))


# speedup vs baseline: 1.0841x; 1.0051x over previous
import functools

import jax
import jax.numpy as jnp
import numpy as np
from jax import lax
from jax.experimental import pallas as pl
from jax.experimental.pallas import tpu as pltpu
from jax.experimental.pallas import tpu_sc as plsc

_NUM_BINS = 16
_GAMMA = 0.01
_HIDDEN = 32
_K = 9
_B, _C, _H, _W = 4, 3, 512, 512
_NIMG = _B * _C
_PIX_PER_IMG = _H * _W
_ROWS = _NIMG * _H
_NWORKERS = 32
_TROWS = 32
_TILE = _TROWS * _W
_NSUB = _ROWS // (_NWORKERS * _TROWS)
_LANES = 16
_TAB = 2 * _NIMG * (_K - 1)

_INV_DT = np.float32(1.0 / (0.125 + 1e-8))


def _hist_mlp_body(x_ref, w1_ref, b1_ref, w2_ref, b2_ref, tab_ref, hist_ref,
                   acc_ref):
    b = pl.program_id(0)
    inv_g = 1.0 / _GAMMA
    zc = 2.0 * inv_g / (_NUM_BINS - 1)
    half = _NUM_BINS // 2
    cr = 64
    ones_row = jnp.ones((1, cr), jnp.bfloat16)
    for sub in range(4):
        xb = x_ref[0, sub * _H:(sub + 1) * _H, :]
        for ch in range(_H // cr):
            xs32 = xb[ch * cr:(ch + 1) * cr, :]
            for lo in (True, False):
                xsv = xs32 if lo else 1.0 - xs32
                p = jnp.exp(xsv * xsv * (-inv_g))
                z = jnp.exp(xsv * zc)
                zb = z.astype(jnp.bfloat16)
                w = p.astype(jnp.bfloat16)
                for k in range(half):
                    if k:
                        w = w * zb
                    row = lax.dot_general(ones_row, w,
                                          (((1,), (0,)), ((), ())),
                                          preferred_element_type=jnp.float32)
                    tgt = k if lo else _NUM_BINS - 1 - k
                    if ch == 0:
                        acc_ref[pl.ds(tgt, 1), :] = row
                    else:
                        acc_ref[pl.ds(tgt, 1), :] = (
                            acc_ref[pl.ds(tgt, 1), :] + row)
        acc = acc_ref[...]
        cc = lax.broadcasted_iota(jnp.int32, (_NUM_BINS, 1), 0).astype(
            jnp.float32) * (1.0 / (_NUM_BINS - 1))
        cc = jnp.minimum(cc, 1.0 - cc)
        amp = jnp.exp(cc * cc * (-inv_g))
        hrow = (jnp.sum(acc * amp, axis=1) * (1.0 / _PIX_PER_IMG)).reshape(
            1, _NUM_BINS)
        hist_ref[pl.ds(4 * b + sub, 1), :] = hrow

    @pl.when(b == _NIMG // 4 - 1)
    def _():
        h = hist_ref[...]
        hid = lax.dot_general(h, w1_ref[...], (((1,), (1,)), ((), ())),
                              preferred_element_type=jnp.float32)
        hid = jnp.maximum(hid + b1_ref[...], 0.0)
        raw = lax.dot_general(hid, w2_ref[...], (((1,), (1,)), ((), ())),
                              preferred_element_type=jnp.float32)
        raw = raw + b2_ref[...]
        sp = jnp.maximum(raw, 0.0) + jnp.log1p(jnp.exp(-jnp.abs(raw)))
        inc = sp + 0.001
        inc = inc / jnp.sum(inc, axis=1, keepdims=True)
        ii = lax.broadcasted_iota(jnp.int32, (_K - 1, _K - 1), 0)
        jj = lax.broadcasted_iota(jnp.int32, (_K - 1, _K - 1), 1)
        excl = (jj > ii).astype(jnp.float32)
        klo = lax.dot_general(inc, excl, (((1,), (0,)), ((), ())),
                              preferred_element_type=jnp.float32)
        t_lo = lax.broadcasted_iota(jnp.int32, (1, _K - 1), 1).astype(
            jnp.float32) * 0.125
        slope = inc * _INV_DT
        icept = klo - t_lo * slope
        both = jnp.concatenate([slope, icept], axis=0)
        for rr in range(2 * _NIMG):
            tab_ref[pl.ds(rr * (_K - 1), _K - 1)] = both[rr]


def _hist_mlp_tc(x12, W1, b1r, W2, b2r):
    return pl.pallas_call(
        _hist_mlp_body,
        grid=(_NIMG // 4,),
        in_specs=[
            pl.BlockSpec((1, 4 * _H, _W), lambda b: (b, 0, 0)),
            pl.BlockSpec((_HIDDEN, _NUM_BINS), lambda b: (0, 0)),
            pl.BlockSpec((1, _HIDDEN), lambda b: (0, 0)),
            pl.BlockSpec((_K - 1, _HIDDEN), lambda b: (0, 0)),
            pl.BlockSpec((1, _K - 1), lambda b: (0, 0)),
        ],
        out_specs=pl.BlockSpec((_TAB,), lambda b: (0,)),
        out_shape=jax.ShapeDtypeStruct((_TAB,), jnp.float32),
        scratch_shapes=[pltpu.VMEM((_NIMG, _NUM_BINS), jnp.float32),
                        pltpu.VMEM((_NUM_BINS, _W), jnp.float32)],
    )(x12, W1, b1r, W2, b2r)


@functools.cache
def _make_lut_sc():
    return functools.partial(
        pl.kernel,
        out_type=jax.ShapeDtypeStruct((_ROWS, _W), jnp.float32),
        mesh=plsc.VectorSubcoreMesh(core_axis_name="c", subcore_axis_name="s"),
        scratch_types=[
            pltpu.VMEM((_TROWS, _W), jnp.float32),
            pltpu.VMEM((_TROWS, _W), jnp.float32),
            pltpu.VMEM((_TROWS, _W), jnp.float32),
            pltpu.VMEM((_TROWS, _W), jnp.float32),
            pltpu.VMEM((_TAB,), jnp.float32),
            pltpu.SemaphoreType.DMA,
            pltpu.SemaphoreType.DMA,
            pltpu.SemaphoreType.DMA,
            pltpu.SemaphoreType.DMA,
        ],
        compiler_params=pltpu.CompilerParams(needs_layout_passes=False),
    )(_lut_sc_body)


def _lut_sc_body(x_hbm, tab_hbm, out_hbm, xb0, xb1, ob0, ob1, tabv,
                 si0, si1, so0, so1):
    wid = lax.axis_index("s") * 2 + lax.axis_index("c")
    pltpu.sync_copy(tab_hbm, tabv)
    xbs, obs, sis, sos = (xb0, xb1), (ob0, ob1), (si0, si1), (so0, so1)

    def row0(t):
        return (wid * _NSUB + t) * _TROWS

    in_cp = [None] * _NSUB
    out_cp = [None] * _NSUB
    in_cp[0] = pltpu.async_copy(
        x_hbm.at[pl.ds(row0(0), _TROWS), :], xbs[0], sis[0])
    for t in range(_NSUB):
        pb = t % 2
        if t + 1 < _NSUB:
            in_cp[t + 1] = pltpu.async_copy(
                x_hbm.at[pl.ds(row0(t + 1), _TROWS), :], xbs[1 - pb],
                sis[1 - pb])
        in_cp[t].wait()
        if t >= 2:
            out_cp[t - 2].wait()
        xb, ob = xbs[pb], obs[pb]
        img8 = lax.shift_right_logical(wid * _NSUB + t, 4) * (_K - 1)
        base = jnp.zeros((_LANES,), jnp.int32) + img8

        @plsc.parallel_loop(0, _TILE // _LANES, unroll=8)
        def body(i):
            r = lax.shift_right_logical(i, 5)
            c = (i & 31) * _LANES
            xv = xb[r, pl.ds(c, _LANES)]
            y = xv * 8.0
            idx = jnp.clip(y.astype(jnp.int32), 0, _K - 2)
            flat = base + idx
            s = plsc.load_gather(tabv, [flat])
            ic = plsc.load_gather(tabv, [flat + _NIMG * (_K - 1)])
            ob[r, pl.ds(c, _LANES)] = ic + s * xv
        out_cp[t] = pltpu.async_copy(
            ob, out_hbm.at[pl.ds(row0(t), _TROWS), :], sos[pb])
    out_cp[_NSUB - 2].wait()
    out_cp[_NSUB - 1].wait()


def kernel(x, W1, b1, W2, b2):
    x12 = x.reshape(_NIMG // 4, 4 * _H, _W)
    tab = _hist_mlp_tc(x12, W1, b1.reshape(1, _HIDDEN), W2,
                       b2.reshape(1, _K - 1))
    y = _make_lut_sc()(x.reshape(_ROWS, _W), tab)
    return (y.reshape(_B, _C, _H, _W), jnp.zeros(()))
